# baseline (device time: 49144 ns/iter reference)
import jax
import jax.numpy as jnp
from jax import lax
from jax.experimental import pallas as pl
from jax.experimental.pallas import tpu as pltpu

N_LAYERS = 3


def kernel(x, Win0, Wout0, Win1, Wout1, Win2, Wout2):
    b, d_y = x.shape
    _, h_x = Win0.shape

    def body(x_ref, win0_ref, wout0_ref, win1_ref, wout1_ref, win2_ref,
             wout2_ref, out_ref,
             ysend_ref, yrecv_ref, xsend_ref, xrecv_ref,
             ysend_sem, yrecv_sems, xsend_sem, xrecv_sems):
        my_x = lax.axis_index("x")
        my_y = lax.axis_index("y")
        y_nbr = (my_x, 1 - my_y)
        x_nbr = (1 - my_x, my_y)

        barrier_sem = pltpu.get_barrier_semaphore()
        for nbr in (y_nbr, x_nbr):
            pl.semaphore_signal(barrier_sem, inc=1, device_id=nbr,
                                device_id_type=pl.DeviceIdType.MESH)
        pl.semaphore_wait(barrier_sem, 2)

        wins = [win0_ref, win1_ref, win2_ref]
        wouts = [wout0_ref, wout1_ref, wout2_ref]

        xb = x_ref[:, :].astype(jnp.bfloat16)
        for l in range(N_LAYERS):
            p = jnp.dot(xb, wins[l][:, :].astype(jnp.bfloat16),
                        preferred_element_type=jnp.float32)
            ysend_ref[:, :] = p.astype(jnp.bfloat16)
            rdma = pltpu.make_async_remote_copy(
                src_ref=ysend_ref, dst_ref=yrecv_ref.at[l],
                send_sem=ysend_sem, recv_sem=yrecv_sems.at[l],
                device_id=y_nbr, device_id_type=pl.DeviceIdType.MESH)
            rdma.start()
            rdma.wait()
            h = p + yrecv_ref[l, :, :].astype(jnp.float32)
            hb = jnp.maximum(h, 0.0).astype(jnp.bfloat16)

            q = jnp.dot(hb, wouts[l][:, :].astype(jnp.bfloat16),
                        preferred_element_type=jnp.float32)
            xsend_ref[:, :] = q.astype(jnp.bfloat16)
            rdma = pltpu.make_async_remote_copy(
                src_ref=xsend_ref, dst_ref=xrecv_ref.at[l],
                send_sem=xsend_sem, recv_sem=xrecv_sems.at[l],
                device_id=x_nbr, device_id_type=pl.DeviceIdType.MESH)
            rdma.start()
            rdma.wait()
            xf = q + xrecv_ref[l, :, :].astype(jnp.float32)
            if l == N_LAYERS - 1:
                out_ref[:, :] = xf
            else:
                xb = xf.astype(jnp.bfloat16)

    return pl.pallas_call(
        body,
        out_shape=jax.ShapeDtypeStruct((b, d_y), jnp.float32),
        in_specs=[pl.BlockSpec(memory_space=pltpu.VMEM)] * 7,
        out_specs=pl.BlockSpec(memory_space=pltpu.VMEM),
        scratch_shapes=[
            pltpu.VMEM((b, h_x), jnp.bfloat16),
            pltpu.VMEM((N_LAYERS, b, h_x), jnp.bfloat16),
            pltpu.VMEM((b, d_y), jnp.bfloat16),
            pltpu.VMEM((N_LAYERS, b, d_y), jnp.bfloat16),
            pltpu.SemaphoreType.DMA,
            pltpu.SemaphoreType.DMA((N_LAYERS,)),
            pltpu.SemaphoreType.DMA,
            pltpu.SemaphoreType.DMA((N_LAYERS,)),
        ],
        compiler_params=pltpu.CompilerParams(collective_id=0),
    )(x, Win0, Wout0, Win1, Wout1, Win2, Wout2)


# device time: 38651 ns/iter; 1.2715x vs baseline; 1.2715x over previous
import jax
import jax.numpy as jnp
from jax import lax
from jax.experimental import pallas as pl
from jax.experimental.pallas import tpu as pltpu

N_LAYERS = 3
C = 2


def kernel(x, Win0, Wout0, Win1, Wout1, Win2, Wout2):
    b, d_y = x.shape
    _, h_x = Win0.shape
    bc = b // C

    def body(x_ref, win0_ref, wout0_ref, win1_ref, wout1_ref, win2_ref,
             wout2_ref, out_ref,
             ysend_ref, yrecv_ref, xsend_ref, xrecv_ref,
             ysend_sems, yrecv_sems, xsend_sems, xrecv_sems):
        my_x = lax.axis_index("x")
        my_y = lax.axis_index("y")
        y_nbr = (my_x, 1 - my_y)
        x_nbr = (1 - my_x, my_y)

        barrier_sem = pltpu.get_barrier_semaphore()
        for nbr in (y_nbr, x_nbr):
            pl.semaphore_signal(barrier_sem, inc=1, device_id=nbr,
                                device_id_type=pl.DeviceIdType.MESH)
        pl.semaphore_wait(barrier_sem, 2)

        wins = [win0_ref, win1_ref, win2_ref]
        wouts = [wout0_ref, wout1_ref, wout2_ref]

        y_rd = [None] * C
        x_rd = [None] * C
        q = [None] * C

        for l in range(N_LAYERS):
            wb = wins[l][:, :].astype(jnp.bfloat16)
            p = [None] * C
            for c in range(C):
                if l == 0:
                    xb_c = x_ref[pl.ds(c * bc, bc), :].astype(jnp.bfloat16)
                else:
                    x_rd[c].wait_recv()
                    x_rd[c].wait_send()
                    xb_c = (q[c] + xrecv_ref[l - 1, c].astype(jnp.float32)
                            ).astype(jnp.bfloat16)
                p[c] = jnp.dot(xb_c, wb, preferred_element_type=jnp.float32)
                ysend_ref[c] = p[c].astype(jnp.bfloat16)
                rd = pltpu.make_async_remote_copy(
                    src_ref=ysend_ref.at[c], dst_ref=yrecv_ref.at[l, c],
                    send_sem=ysend_sems.at[c], recv_sem=yrecv_sems.at[l, c],
                    device_id=y_nbr, device_id_type=pl.DeviceIdType.MESH)
                rd.start()
                y_rd[c] = rd

            wob = wouts[l][:, :].astype(jnp.bfloat16)
            for c in range(C):
                y_rd[c].wait_recv()
                y_rd[c].wait_send()
                h = jnp.maximum(p[c] + yrecv_ref[l, c].astype(jnp.float32),
                                0.0).astype(jnp.bfloat16)
                q[c] = jnp.dot(h, wob, preferred_element_type=jnp.float32)
                xsend_ref[c] = q[c].astype(jnp.bfloat16)
                rd = pltpu.make_async_remote_copy(
                    src_ref=xsend_ref.at[c], dst_ref=xrecv_ref.at[l, c],
                    send_sem=xsend_sems.at[c], recv_sem=xrecv_sems.at[l, c],
                    device_id=x_nbr, device_id_type=pl.DeviceIdType.MESH)
                rd.start()
                x_rd[c] = rd

        for c in range(C):
            x_rd[c].wait_recv()
            x_rd[c].wait_send()
            out_ref[pl.ds(c * bc, bc), :] = (
                q[c] + xrecv_ref[N_LAYERS - 1, c].astype(jnp.float32))

    return pl.pallas_call(
        body,
        out_shape=jax.ShapeDtypeStruct((b, d_y), jnp.float32),
        in_specs=[pl.BlockSpec(memory_space=pltpu.VMEM)] * 7,
        out_specs=pl.BlockSpec(memory_space=pltpu.VMEM),
        scratch_shapes=[
            pltpu.VMEM((C, bc, h_x), jnp.bfloat16),
            pltpu.VMEM((N_LAYERS, C, bc, h_x), jnp.bfloat16),
            pltpu.VMEM((C, bc, d_y), jnp.bfloat16),
            pltpu.VMEM((N_LAYERS, C, bc, d_y), jnp.bfloat16),
            pltpu.SemaphoreType.DMA((C,)),
            pltpu.SemaphoreType.DMA((N_LAYERS, C)),
            pltpu.SemaphoreType.DMA((C,)),
            pltpu.SemaphoreType.DMA((N_LAYERS, C)),
        ],
        compiler_params=pltpu.CompilerParams(collective_id=0),
    )(x, Win0, Wout0, Win1, Wout1, Win2, Wout2)


# device time: 36957 ns/iter; 1.3298x vs baseline; 1.0458x over previous
import jax
import jax.numpy as jnp
from jax import lax
from jax.experimental import pallas as pl
from jax.experimental.pallas import tpu as pltpu

N_LAYERS = 3
C = 4


def kernel(x, Win0, Wout0, Win1, Wout1, Win2, Wout2):
    b, d_y = x.shape
    _, h_x = Win0.shape
    bc = b // C

    def body(x_ref, win0_ref, wout0_ref, win1_ref, wout1_ref, win2_ref,
             wout2_ref, out_ref,
             ysend_ref, yrecv_ref, xsend_ref, xrecv_ref,
             ysend_sems, yrecv_sems, xsend_sems, xrecv_sems):
        my_x = lax.axis_index("x")
        my_y = lax.axis_index("y")
        y_nbr = (my_x, 1 - my_y)
        x_nbr = (1 - my_x, my_y)

        barrier_sem = pltpu.get_barrier_semaphore()
        for nbr in (y_nbr, x_nbr):
            pl.semaphore_signal(barrier_sem, inc=1, device_id=nbr,
                                device_id_type=pl.DeviceIdType.MESH)
        pl.semaphore_wait(barrier_sem, 2)

        wins = [win0_ref, win1_ref, win2_ref]
        wouts = [wout0_ref, wout1_ref, wout2_ref]

        y_rd = [None] * C
        x_rd = [None] * C
        q = [None] * C

        for l in range(N_LAYERS):
            wb = wins[l][:, :].astype(jnp.bfloat16)
            p = [None] * C
            for c in range(C):
                if l == 0:
                    xb_c = x_ref[pl.ds(c * bc, bc), :].astype(jnp.bfloat16)
                else:
                    x_rd[c].wait_recv()
                    x_rd[c].wait_send()
                    xb_c = (q[c] + xrecv_ref[l - 1, c].astype(jnp.float32)
                            ).astype(jnp.bfloat16)
                p[c] = jnp.dot(xb_c, wb, preferred_element_type=jnp.float32)
                ysend_ref[c] = p[c].astype(jnp.bfloat16)
                rd = pltpu.make_async_remote_copy(
                    src_ref=ysend_ref.at[c], dst_ref=yrecv_ref.at[l, c],
                    send_sem=ysend_sems.at[c], recv_sem=yrecv_sems.at[l, c],
                    device_id=y_nbr, device_id_type=pl.DeviceIdType.MESH)
                rd.start()
                y_rd[c] = rd

            wob = wouts[l][:, :].astype(jnp.bfloat16)
            for c in range(C):
                y_rd[c].wait_recv()
                y_rd[c].wait_send()
                h = jnp.maximum(p[c] + yrecv_ref[l, c].astype(jnp.float32),
                                0.0).astype(jnp.bfloat16)
                q[c] = jnp.dot(h, wob, preferred_element_type=jnp.float32)
                xsend_ref[c] = q[c].astype(jnp.bfloat16)
                rd = pltpu.make_async_remote_copy(
                    src_ref=xsend_ref.at[c], dst_ref=xrecv_ref.at[l, c],
                    send_sem=xsend_sems.at[c], recv_sem=xrecv_sems.at[l, c],
                    device_id=x_nbr, device_id_type=pl.DeviceIdType.MESH)
                rd.start()
                x_rd[c] = rd

        for c in range(C):
            x_rd[c].wait_recv()
            x_rd[c].wait_send()
            out_ref[pl.ds(c * bc, bc), :] = (
                q[c] + xrecv_ref[N_LAYERS - 1, c].astype(jnp.float32))

    return pl.pallas_call(
        body,
        out_shape=jax.ShapeDtypeStruct((b, d_y), jnp.float32),
        in_specs=[pl.BlockSpec(memory_space=pltpu.VMEM)] * 7,
        out_specs=pl.BlockSpec(memory_space=pltpu.VMEM),
        scratch_shapes=[
            pltpu.VMEM((C, bc, h_x), jnp.bfloat16),
            pltpu.VMEM((N_LAYERS, C, bc, h_x), jnp.bfloat16),
            pltpu.VMEM((C, bc, d_y), jnp.bfloat16),
            pltpu.VMEM((N_LAYERS, C, bc, d_y), jnp.bfloat16),
            pltpu.SemaphoreType.DMA((C,)),
            pltpu.SemaphoreType.DMA((N_LAYERS, C)),
            pltpu.SemaphoreType.DMA((C,)),
            pltpu.SemaphoreType.DMA((N_LAYERS, C)),
        ],
        compiler_params=pltpu.CompilerParams(collective_id=0),
    )(x, Win0, Wout0, Win1, Wout1, Win2, Wout2)
